# baseline (device time: 102444 ns/iter reference)
import functools

import jax
import jax.numpy as jnp
from jax import lax
from jax.experimental import pallas as pl
from jax.experimental.pallas import tpu as pltpu

N_DEV = 16
N_EXP = 64
E_LOCAL = N_EXP // N_DEV
CAP = 12
SLOTS = E_LOCAL * CAP
N_TOK = 1024
D_MODEL = 512
D_OUT = 1024


def kernel(x, router_W, route_idx, expert_W):
    del router_W

    e = route_idx[:, 0]
    oh = (e[:, None] == jnp.arange(N_EXP, dtype=e.dtype)).astype(jnp.int32)
    rank = jnp.take_along_axis(jnp.cumsum(oh, axis=0) - oh, e[:, None], axis=1)[:, 0]
    kept = rank < CAP
    g = e * CAP + rank

    my = lax.axis_index("i")
    local_slot_ids = jnp.arange(SLOTS, dtype=g.dtype) + my.astype(g.dtype) * SLOTS
    D = ((g[None, :] == local_slot_ids[:, None]) & kept[None, :]).astype(jnp.float32)
    C = ((g[:, None] == jnp.arange(N_EXP * CAP, dtype=g.dtype)[None, :])
         & kept[:, None]).astype(jnp.float32)

    def body(x_ref, w_ref, d_ref, c_ref, out_ref, yall_ref, send_sems, recv_sems):
        my_pos = lax.axis_index("i")
        left = lax.rem(my_pos - 1 + N_DEV, N_DEV)
        right = lax.rem(my_pos + 1, N_DEV)

        barrier_sem = pltpu.get_barrier_semaphore()
        for nbr in (left, right):
            pl.semaphore_signal(barrier_sem, inc=1, device_id=(nbr,),
                                device_id_type=pl.DeviceIdType.MESH)
        pl.semaphore_wait(barrier_sem, 2)

        xg = jnp.dot(d_ref[:, :], x_ref[:, :],
                     preferred_element_type=jnp.float32)
        y = jnp.concatenate(
            [jnp.dot(xg[le * CAP:(le + 1) * CAP, :], w_ref[le, :, :],
                     preferred_element_type=jnp.float32)
             for le in range(E_LOCAL)],
            axis=0,
        )
        yall_ref[pl.ds(my_pos, 1), :, :] = y[None, :, :]

        for h in range(N_DEV - 1):
            src_orig = lax.rem(my_pos - h + N_DEV, N_DEV)
            rdma = pltpu.make_async_remote_copy(
                src_ref=yall_ref.at[src_orig],
                dst_ref=yall_ref.at[src_orig],
                send_sem=send_sems.at[h],
                recv_sem=recv_sems.at[h],
                device_id=(right,),
                device_id_type=pl.DeviceIdType.MESH,
            )
            rdma.start()
            rdma.wait()

        ya = yall_ref[:, :, :].reshape(N_DEV * SLOTS, D_OUT)
        out_ref[:, :] = jnp.dot(c_ref[:, :], ya,
                                preferred_element_type=jnp.float32)

        @functools.partial(pl.run_scoped,
                           second_barrier=pltpu.SemaphoreType.REGULAR)
        def _(second_barrier):
            for nbr in (left, right):
                pl.semaphore_signal(second_barrier, inc=1, device_id=(nbr,),
                                    device_id_type=pl.DeviceIdType.MESH)
            pl.semaphore_wait(second_barrier, 2)

    return pl.pallas_call(
        body,
        out_shape=jax.ShapeDtypeStruct((N_TOK, D_OUT), jnp.float32),
        in_specs=[
            pl.BlockSpec(memory_space=pltpu.VMEM),
            pl.BlockSpec(memory_space=pltpu.VMEM),
            pl.BlockSpec(memory_space=pltpu.VMEM),
            pl.BlockSpec(memory_space=pltpu.VMEM),
        ],
        out_specs=pl.BlockSpec(memory_space=pltpu.VMEM),
        scratch_shapes=[
            pltpu.VMEM((N_DEV, SLOTS, D_OUT), jnp.float32),
            pltpu.SemaphoreType.DMA((N_DEV - 1,)),
            pltpu.SemaphoreType.DMA((N_DEV - 1,)),
        ],
        compiler_params=pltpu.CompilerParams(collective_id=0),
    )(x, expert_W, D, C)


# device time: 57591 ns/iter; 1.7788x vs baseline; 1.7788x over previous
import functools

import jax
import jax.numpy as jnp
from jax import lax
from jax.experimental import pallas as pl
from jax.experimental.pallas import tpu as pltpu

N_DEV = 16
N_EXP = 64
E_LOCAL = N_EXP // N_DEV
CAP = 12
SLOTS = E_LOCAL * CAP
N_TOK = 1024
D_MODEL = 512
D_OUT = 1024
R_HOPS = N_DEV // 2
L_HOPS = N_DEV // 2 - 1


def kernel(x, router_W, route_idx, expert_W):
    del router_W
    route_idx_row = route_idx.reshape(1, N_TOK)

    def body(x_ref, w_ref, idx_ref, out_ref, yall_ref,
             send_r, recv_r, send_l, recv_l):
        my_pos = lax.axis_index("i")
        left = lax.rem(my_pos - 1 + N_DEV, N_DEV)
        right = lax.rem(my_pos + 1, N_DEV)

        barrier_sem = pltpu.get_barrier_semaphore()
        for nbr in (left, right):
            pl.semaphore_signal(barrier_sem, inc=1, device_id=(nbr,),
                                device_id_type=pl.DeviceIdType.MESH)
        pl.semaphore_wait(barrier_sem, 2)

        e = idx_ref[:, :]
        oh = (lax.broadcasted_iota(jnp.int32, (N_EXP, N_TOK), 0)
              == e).astype(jnp.float32)
        triu = (lax.broadcasted_iota(jnp.int32, (N_TOK, N_TOK), 0)
                < lax.broadcasted_iota(jnp.int32, (N_TOK, N_TOK), 1)
                ).astype(jnp.float32)
        rank_mat = jnp.dot(oh, triu, preferred_element_type=jnp.float32)
        rank = jnp.sum(rank_mat * oh, axis=0, keepdims=True)
        kept = rank < float(CAP)
        rank_i = rank.astype(jnp.int32)
        d_orig = e // E_LOCAL
        s_within = (e % E_LOCAL) * CAP + rank_i
        rel = lax.rem(my_pos - d_orig + N_DEV, N_DEV)
        j = jnp.where(rel == 0, 0,
                      jnp.where(rel <= R_HOPS, 2 * rel - 1, 32 - 2 * rel))
        gp = jnp.where(kept, j * SLOTS + s_within, -1)
        Cp = (lax.broadcasted_iota(jnp.int32, (N_DEV * SLOTS, N_TOK), 0)
              == gp).astype(jnp.float32)

        xg = lax.dot_general(Cp[0:SLOTS, :], x_ref[:, :],
                             (((1,), (0,)), ((), ())),
                             preferred_element_type=jnp.float32)
        y = jnp.concatenate(
            [jnp.dot(xg[le * CAP:(le + 1) * CAP, :], w_ref[le, :, :],
                     preferred_element_type=jnp.float32)
             for le in range(E_LOCAL)],
            axis=0,
        )
        yall_ref[0, :, :] = y

        def combine(lo_slot, hi_slot):
            blk = yall_ref[lo_slot:hi_slot, :, :].reshape(
                (hi_slot - lo_slot) * SLOTS, D_OUT)
            return lax.dot_general(
                Cp[lo_slot * SLOTS:hi_slot * SLOTS, :], blk,
                (((0,), (0,)), ((), ())),
                preferred_element_type=jnp.float32)

        sends = []
        acc = None
        for h in range(R_HOPS):
            rd_r = pltpu.make_async_remote_copy(
                src_ref=yall_ref.at[0 if h == 0 else 2 * h - 1],
                dst_ref=yall_ref.at[2 * h + 1],
                send_sem=send_r.at[h],
                recv_sem=recv_r.at[h],
                device_id=(right,),
                device_id_type=pl.DeviceIdType.MESH,
            )
            rd_r.start()
            sends.append(rd_r)
            if h < L_HOPS:
                rd_l = pltpu.make_async_remote_copy(
                    src_ref=yall_ref.at[0 if h == 0 else 2 * h],
                    dst_ref=yall_ref.at[2 * h + 2],
                    send_sem=send_l.at[h],
                    recv_sem=recv_l.at[h],
                    device_id=(left,),
                    device_id_type=pl.DeviceIdType.MESH,
                )
                rd_l.start()
                sends.append(rd_l)
            if h == 0:
                acc = lax.dot_general(Cp[0:SLOTS, :], y,
                                      (((0,), (0,)), ((), ())),
                                      preferred_element_type=jnp.float32)
            else:
                acc = acc + combine(2 * h - 1, 2 * h + 1)
            rd_r.wait_recv()
            if h < L_HOPS:
                rd_l.wait_recv()
        acc = acc + combine(2 * R_HOPS - 1, 2 * R_HOPS)
        out_ref[:, :] = acc

        for s in sends:
            s.wait_send()

        @functools.partial(pl.run_scoped,
                           second_barrier=pltpu.SemaphoreType.REGULAR)
        def _(second_barrier):
            for nbr in (left, right):
                pl.semaphore_signal(second_barrier, inc=1, device_id=(nbr,),
                                    device_id_type=pl.DeviceIdType.MESH)
            pl.semaphore_wait(second_barrier, 2)

    return pl.pallas_call(
        body,
        out_shape=jax.ShapeDtypeStruct((N_TOK, D_OUT), jnp.float32),
        in_specs=[
            pl.BlockSpec(memory_space=pltpu.VMEM),
            pl.BlockSpec(memory_space=pltpu.VMEM),
            pl.BlockSpec(memory_space=pltpu.VMEM),
        ],
        out_specs=pl.BlockSpec(memory_space=pltpu.VMEM),
        scratch_shapes=[
            pltpu.VMEM((N_DEV, SLOTS, D_OUT), jnp.float32),
            pltpu.SemaphoreType.DMA((R_HOPS,)),
            pltpu.SemaphoreType.DMA((R_HOPS,)),
            pltpu.SemaphoreType.DMA((L_HOPS,)),
            pltpu.SemaphoreType.DMA((L_HOPS,)),
        ],
        compiler_params=pltpu.CompilerParams(collective_id=0),
    )(x, expert_W, route_idx_row)


# device time: 33151 ns/iter; 3.0902x vs baseline; 1.7372x over previous
import os

import jax
import jax.numpy as jnp
from jax import lax
from jax.experimental import pallas as pl
from jax.experimental.pallas import tpu as pltpu

N_DEV = 16
N_EXP = 64
E_LOCAL = N_EXP // N_DEV
CAP = 12
SLOTS = E_LOCAL * CAP
N_TOK = 1024
D_MODEL = 512
D_OUT = 1024
N_P = 4
N_Q = 4
NSLOT = N_Q * (2 * N_P - 1)

_KVAR = os.environ.get("KVAR", "")

_DZS = (-3, -2, -1, 1, 2, 3)


def kernel(x, router_W, route_idx, expert_W):
    del router_W
    route_idx_row = route_idx.reshape(1, N_TOK)

    def body(x_ref, w_ref, idx_ref, out_ref, yall_ref,
             sz, rz, spr, rpl, spl, rpr, spd, rpd):
        my_pos = lax.axis_index("i")
        q = lax.rem(my_pos, N_Q)
        p = my_pos // N_Q
        dev_right = N_P * p + lax.rem(q + 1, N_Q)
        dev_left = N_P * p + lax.rem(q + 3, N_Q)
        dev_diag = N_P * p + lax.rem(q + 2, N_Q)

        def zvalid(dz):
            pz = p + dz
            return jnp.logical_and(pz >= 0, pz <= N_P - 1)

        yall_ref[:, :, :] = jnp.zeros((NSLOT, SLOTS, D_OUT), jnp.bfloat16)

        barrier_sem = pltpu.get_barrier_semaphore()
        for pz in range(N_P):
            @pl.when(pz != p)
            def _():
                pl.semaphore_signal(barrier_sem, inc=1,
                                    device_id=(N_P * pz + q,),
                                    device_id_type=pl.DeviceIdType.MESH)
        for dev in (dev_right, dev_left, dev_diag):
            pl.semaphore_signal(barrier_sem, inc=1, device_id=(dev,),
                                device_id_type=pl.DeviceIdType.MESH)

        e = idx_ref[:, :]
        oh = (lax.broadcasted_iota(jnp.int32, (N_EXP, N_TOK), 0)
              == e).astype(jnp.bfloat16)
        triu = (lax.broadcasted_iota(jnp.int32, (N_TOK, N_TOK), 0)
                < lax.broadcasted_iota(jnp.int32, (N_TOK, N_TOK), 1)
                ).astype(jnp.bfloat16)
        rank_mat = jnp.dot(oh, triu, preferred_element_type=jnp.float32)
        rank = jnp.sum(rank_mat * oh.astype(jnp.float32), axis=0,
                       keepdims=True)
        kept = rank < float(CAP)
        rank_i = rank.astype(jnp.int32)
        d_orig = e // E_LOCAL
        s_within = (e % E_LOCAL) * CAP + rank_i
        q_o = lax.rem(d_orig, N_Q)
        p_o = d_orig // N_Q
        cq = lax.rem(q - q_o + N_Q, N_Q)
        dz = p_o - p
        gp = jnp.where(kept, (cq * (2 * N_P - 1) + dz + 3) * SLOTS + s_within,
                       -1)

        d_local = (lax.broadcasted_iota(jnp.int32, (SLOTS, N_TOK), 0)
                   == gp - 3 * SLOTS).astype(jnp.float32)
        xg = jnp.dot(d_local, x_ref[:, :],
                     preferred_element_type=jnp.float32)
        y = jnp.concatenate(
            [jnp.dot(xg[le * CAP:(le + 1) * CAP, :], w_ref[le, :, :],
                     preferred_element_type=jnp.float32)
             for le in range(E_LOCAL)],
            axis=0,
        )
        yall_ref[3, :, :] = y.astype(jnp.bfloat16)

        ring = _KVAR != "noring"
        pl.semaphore_wait(barrier_sem, 6)

        def plane_sends(dzs):
            s = dzs + 3
            for dst_cq, sem_s, sem_r, dev in (
                    (1, spr, rpl, dev_right),
                    (3, spl, rpr, dev_left),
                    (2, spd, rpd, dev_diag)):
                rdma = pltpu.make_async_remote_copy(
                    src_ref=yall_ref.at[s],
                    dst_ref=yall_ref.at[dst_cq * (2 * N_P - 1) + s],
                    send_sem=sem_s.at[s],
                    recv_sem=sem_r.at[s],
                    device_id=(dev,),
                    device_id_type=pl.DeviceIdType.MESH,
                )
                rdma.start()

        def recv_wait(arr_slot, sem, idx):
            pltpu.make_async_remote_copy(
                src_ref=yall_ref.at[arr_slot],
                dst_ref=yall_ref.at[arr_slot],
                send_sem=sz.at[idx],
                recv_sem=sem.at[idx],
                device_id=(dev_right,),
                device_id_type=pl.DeviceIdType.MESH,
            ).wait_recv()

        if ring:
            for dzt in _DZS:
                @pl.when(zvalid(dzt))
                def _():
                    rdma = pltpu.make_async_remote_copy(
                        src_ref=yall_ref.at[3],
                        dst_ref=yall_ref.at[-dzt + 3],
                        send_sem=sz.at[dzt + 3],
                        recv_sem=rz.at[-dzt + 3],
                        device_id=(N_P * (p + dzt) + q,),
                        device_id_type=pl.DeviceIdType.MESH,
                    )
                    rdma.start()
            plane_sends(0)

        Cp = (lax.broadcasted_iota(jnp.int32, (NSLOT * SLOTS, N_TOK), 0)
              == gp).astype(jnp.bfloat16)

        def combine(lo_slot, hi_slot):
            blk = yall_ref[lo_slot:hi_slot, :, :].reshape(
                (hi_slot - lo_slot) * SLOTS, D_OUT)
            return lax.dot_general(
                Cp[lo_slot * SLOTS:hi_slot * SLOTS, :], blk,
                (((0,), (0,)), ((), ())),
                preferred_element_type=jnp.float32)

        if ring:
            for dzr in _DZS:
                @pl.when(zvalid(dzr))
                def _():
                    recv_wait(dzr + 3, rz, dzr + 3)
                    plane_sends(dzr)

        acc = combine(0, 2 * N_P - 1)

        if ring:
            for cq_a, sem in ((1, rpl), (3, rpr), (2, rpd)):
                for dzr in (-3, -2, -1, 0, 1, 2, 3):
                    @pl.when(zvalid(dzr))
                    def _():
                        recv_wait(cq_a * (2 * N_P - 1) + dzr + 3, sem,
                                  dzr + 3)
            acc = acc + combine(2 * N_P - 1, NSLOT)
        out_ref[:, :] = acc

        if ring:
            for dzt in _DZS:
                @pl.when(zvalid(dzt))
                def _():
                    pltpu.make_async_remote_copy(
                        src_ref=yall_ref.at[3],
                        dst_ref=yall_ref.at[-dzt + 3],
                        send_sem=sz.at[dzt + 3],
                        recv_sem=rz.at[-dzt + 3],
                        device_id=(N_P * (p + dzt) + q,),
                        device_id_type=pl.DeviceIdType.MESH,
                    ).wait_send()
            for dzs in (-3, -2, -1, 0, 1, 2, 3):
                @pl.when(zvalid(dzs))
                def _():
                    for dst_cq, sem_s, sem_r, dev in (
                            (1, spr, rpl, dev_right),
                            (3, spl, rpr, dev_left),
                            (2, spd, rpd, dev_diag)):
                        pltpu.make_async_remote_copy(
                            src_ref=yall_ref.at[dzs + 3],
                            dst_ref=yall_ref.at[dst_cq * (2 * N_P - 1)
                                                + dzs + 3],
                            send_sem=sem_s.at[dzs + 3],
                            recv_sem=sem_r.at[dzs + 3],
                            device_id=(dev,),
                            device_id_type=pl.DeviceIdType.MESH,
                        ).wait_send()

    return pl.pallas_call(
        body,
        out_shape=jax.ShapeDtypeStruct((N_TOK, D_OUT), jnp.float32),
        in_specs=[
            pl.BlockSpec(memory_space=pltpu.VMEM),
            pl.BlockSpec(memory_space=pltpu.VMEM),
            pl.BlockSpec(memory_space=pltpu.VMEM),
        ],
        out_specs=pl.BlockSpec(memory_space=pltpu.VMEM),
        scratch_shapes=[
            pltpu.VMEM((NSLOT, SLOTS, D_OUT), jnp.bfloat16),
            pltpu.SemaphoreType.DMA((7,)),
            pltpu.SemaphoreType.DMA((7,)),
            pltpu.SemaphoreType.DMA((7,)),
            pltpu.SemaphoreType.DMA((7,)),
            pltpu.SemaphoreType.DMA((7,)),
            pltpu.SemaphoreType.DMA((7,)),
            pltpu.SemaphoreType.DMA((7,)),
            pltpu.SemaphoreType.DMA((7,)),
        ],
        compiler_params=pltpu.CompilerParams(
            collective_id=0, vmem_limit_bytes=100 * 1024 * 1024),
    )(x, expert_W, route_idx_row)
